# speedup vs baseline: 1.2556x; 1.2556x over previous
import functools

import jax
import jax.numpy as jnp
from jax import lax
from jax.experimental import pallas as pl
from jax.experimental.pallas import tpu as pltpu
from jax.experimental.pallas import tpu_sc as plsc

DIM = 768
B = 1024 * 50
NW = 32
B_PER_W = B // NW
CHUNK = 80
NCHUNK = B_PER_W // CHUNK

_mesh = plsc.VectorSubcoreMesh(core_axis_name="c", subcore_axis_name="s")


@functools.partial(
    pl.kernel,
    mesh=_mesh,
    out_type=jax.ShapeDtypeStruct((B, DIM), jnp.float32),
    scratch_types=[
        pltpu.VMEM((B_PER_W,), jnp.int32),
        pltpu.VMEM((CHUNK, DIM), jnp.float32),
        pltpu.SemaphoreType.DMA,
    ],
)
def _gather(table_hbm, idx_hbm, out_hbm, idx_v, rows, sem):
    wid = lax.axis_index("s") * 2 + lax.axis_index("c")
    base = wid * B_PER_W
    pltpu.sync_copy(idx_hbm.at[pl.ds(base, B_PER_W)], idx_v)

    def body(c, _):
        off = pl.multiple_of(c * CHUNK, 8)
        pltpu.async_copy(table_hbm.at[idx_v.at[pl.ds(off, CHUNK)]], rows, sem).wait()
        pltpu.sync_copy(rows, out_hbm.at[pl.ds(base + off, CHUNK)])
        return 0

    lax.fori_loop(0, NCHUNK, body, 0)


def kernel(x, embedding_weight):
    idx = x.reshape(-1)
    out = _gather(embedding_weight, idx)
    return out.reshape(x.shape[0], x.shape[1], DIM)


# speedup vs baseline: 1.2781x; 1.0179x over previous
import functools

import jax
import jax.numpy as jnp
from jax import lax
from jax.experimental import pallas as pl
from jax.experimental.pallas import tpu as pltpu
from jax.experimental.pallas import tpu_sc as plsc

DIM = 768
B = 1024 * 50
NW = 32
B_PER_W = B // NW
CHUNK = 80
NCHUNK = B_PER_W // CHUNK

_mesh = plsc.VectorSubcoreMesh(core_axis_name="c", subcore_axis_name="s")


@functools.partial(
    pl.kernel,
    mesh=_mesh,
    out_type=jax.ShapeDtypeStruct((B, DIM), jnp.float32),
    scratch_types=[
        pltpu.VMEM((B_PER_W,), jnp.int32),
        pltpu.VMEM((CHUNK, DIM), jnp.float32),
        pltpu.VMEM((CHUNK, DIM), jnp.float32),
        pltpu.SemaphoreType.DMA,
        pltpu.SemaphoreType.DMA,
        pltpu.SemaphoreType.DMA,
        pltpu.SemaphoreType.DMA,
    ],
)
def _gather(table_hbm, idx_hbm, out_hbm, idx_v, rows0, rows1,
            gsem0, gsem1, osem0, osem1):
    wid = lax.axis_index("s") * 2 + lax.axis_index("c")
    base = wid * B_PER_W
    pltpu.sync_copy(idx_hbm.at[pl.ds(base, B_PER_W)], idx_v)

    bufs = (rows0, rows1)
    gsems = (gsem0, gsem1)
    osems = (osem0, osem1)

    def idx_slice(c):
        return idx_v.at[pl.ds(pl.multiple_of(c * CHUNK, 8), CHUNK)]

    def out_slice(c):
        return out_hbm.at[pl.ds(pl.multiple_of(base + c * CHUNK, 8), CHUNK)]

    def g_start(c, j):
        pltpu.async_copy(table_hbm.at[idx_slice(c)], bufs[j], gsems[j])

    def g_wait(c, j):
        pltpu.make_async_copy(table_hbm.at[idx_slice(c)], bufs[j], gsems[j]).wait()

    def ow_start(c, j):
        pltpu.async_copy(bufs[j], out_slice(c), osems[j])

    def ow_wait(c, j):
        pltpu.make_async_copy(bufs[j], out_slice(c), osems[j]).wait()

    g_start(0, 0)
    g_start(1, 1)

    def body(c2, _):
        c = c2 * 2
        for j in range(2):
            g_wait(c + j, j)
            ow_start(c + j, j)
        for j in range(2):
            ow_wait(c + j, j)
            g_start(c + 2 + j, j)
        return 0

    lax.fori_loop(0, NCHUNK // 2 - 1, body, 0)

    c = NCHUNK - 2
    for j in range(2):
        g_wait(c + j, j)
        ow_start(c + j, j)
    for j in range(2):
        ow_wait(c + j, j)


def kernel(x, embedding_weight):
    idx = x.reshape(-1)
    out = _gather(embedding_weight, idx)
    return out.reshape(x.shape[0], x.shape[1], DIM)
